# Initial kernel scaffold; baseline (speedup 1.0000x reference)
import jax
import jax.numpy as jnp
from jax.experimental import pallas as pl


def kernel(seq, table):
    raise NotImplementedError("write your pallas kernel here")


# speedup vs baseline: 7.7767x; 7.7767x over previous
import functools

import jax
import jax.numpy as jnp
from jax import lax
from jax.experimental import pallas as pl
from jax.experimental.pallas import tpu as pltpu
from jax.experimental.pallas import tpu_sc as plsc

D = 128
B = 1024 * 200
NC, NS = 2, 16
NW = NC * NS
BPW = B // NW
CH = 128
NCH = BPW // CH
NBUF = 5
NOUT = NCH // NBUF

_mesh = plsc.VectorSubcoreMesh(core_axis_name="c", subcore_axis_name="s")


@functools.partial(
    pl.kernel,
    mesh=_mesh,
    out_type=jax.ShapeDtypeStruct((B, D), jnp.float32),
    scratch_types=[
        pltpu.VMEM((BPW,), jnp.int32),
        pltpu.VMEM((NBUF, CH, D), jnp.float32),
        pltpu.SemaphoreType.DMA((NBUF,)),
        pltpu.SemaphoreType.DMA((NBUF,)),
    ],
)
def _embed_gather(idx_hbm, table_hbm, out_hbm, idx_v, rows_v, gsem, psem):
    wid = lax.axis_index("s") * NC + lax.axis_index("c")
    base = wid * BPW
    pltpu.sync_copy(idx_hbm.at[pl.ds(base, BPW)], idx_v)

    def gather(b, g):
        return pltpu.make_async_copy(
            table_hbm.at[idx_v.at[pl.ds(g * CH, CH)]], rows_v.at[b], gsem.at[b]
        )

    def put(b, g):
        return pltpu.make_async_copy(
            rows_v.at[b], out_hbm.at[pl.ds(base + g * CH, CH)], psem.at[b]
        )

    for b in range(NBUF):
        gather(b, b).start()

    def round_body(o, carry):
        for b in range(NBUF):
            g = o * NBUF + b
            gather(b, g).wait()
            put(b, g).start()
        for b in range(NBUF):
            g = o * NBUF + b
            put(b, g).wait()
            gather(b, g + NBUF).start()
        return carry

    lax.fori_loop(0, NOUT - 1, round_body, 0, unroll=False)

    for b in range(NBUF):
        g = (NOUT - 1) * NBUF + b
        gather(b, g).wait()
        put(b, g).start()
    for b in range(NBUF):
        g = (NOUT - 1) * NBUF + b
        put(b, g).wait()


def kernel(seq, table):
    idx = seq.reshape(-1).astype(jnp.int32)
    out = _embed_gather(idx, table)
    return out.reshape(seq.shape + (D,))


# speedup vs baseline: 7.9715x; 1.0250x over previous
import functools

import jax
import jax.numpy as jnp
from jax import lax
from jax.experimental import pallas as pl
from jax.experimental.pallas import tpu as pltpu
from jax.experimental.pallas import tpu_sc as plsc

D = 128
B = 1024 * 200
NC, NS = 2, 16
NW = NC * NS
BPW = B // NW
CH = 64
NCH = BPW // CH
NBUF = 10
NOUT = NCH // NBUF

_mesh = plsc.VectorSubcoreMesh(core_axis_name="c", subcore_axis_name="s")


@functools.partial(
    pl.kernel,
    mesh=_mesh,
    out_type=jax.ShapeDtypeStruct((B, D), jnp.float32),
    scratch_types=[
        pltpu.VMEM((BPW,), jnp.int32),
        pltpu.VMEM((NBUF, CH, D), jnp.float32),
        pltpu.SemaphoreType.DMA((NBUF,)),
        pltpu.SemaphoreType.DMA((NBUF,)),
    ],
)
def _embed_gather(idx_hbm, table_hbm, out_hbm, idx_v, rows_v, gsem, psem):
    wid = lax.axis_index("s") * NC + lax.axis_index("c")
    base = wid * BPW
    pltpu.sync_copy(idx_hbm.at[pl.ds(base, BPW)], idx_v)

    def gather(b, g):
        return pltpu.make_async_copy(
            table_hbm.at[idx_v.at[pl.ds(g * CH, CH)]], rows_v.at[b], gsem.at[b]
        )

    def put(b, g):
        return pltpu.make_async_copy(
            rows_v.at[b], out_hbm.at[pl.ds(base + g * CH, CH)], psem.at[b]
        )

    for b in range(NBUF):
        gather(b, b).start()

    def round_body(o, carry):
        for b in range(NBUF):
            g = o * NBUF + b
            gather(b, g).wait()
            put(b, g).start()
        for b in range(NBUF):
            g = o * NBUF + b
            put(b, g).wait()
            gather(b, g + NBUF).start()
        return carry

    lax.fori_loop(0, NOUT - 1, round_body, 0, unroll=False)

    for b in range(NBUF):
        g = (NOUT - 1) * NBUF + b
        gather(b, g).wait()
        put(b, g).start()
    for b in range(NBUF):
        g = (NOUT - 1) * NBUF + b
        put(b, g).wait()


def kernel(seq, table):
    idx = seq.reshape(-1).astype(jnp.int32)
    out = _embed_gather(idx, table)
    return out.reshape(seq.shape + (D,))
